# baseline (speedup 1.0000x reference)
import functools

import jax
import jax.numpy as jnp
from jax import lax
from jax.experimental import pallas as pl
from jax.experimental.pallas import tpu as pltpu
from jax.experimental.pallas import tpu_sc as plsc

T = 4096
D = 1024
E = 8
TH = T // 2
MBLK = 256
NBH = (2 * TH + E * MBLK) // MBLK
NPADH = NBH * MBLK
TPW = TH // 32
_NEG_INF = float("-inf")


def _gating_body(x_ref, wg_ref, e0_ref, e1_ref, g0_ref, g1_ref, hist_ref):
    x = x_ref[...]
    logits = jnp.dot(x, wg_ref[...], preferred_element_type=jnp.float32)
    m = jnp.max(logits, axis=-1, keepdims=True)
    p = jnp.exp(logits - m)
    p = p / jnp.sum(p, axis=-1, keepdims=True)

    ii = lax.broadcasted_iota(jnp.int32, p.shape, 1)
    m1 = jnp.max(p, axis=-1, keepdims=True)
    i1 = jnp.min(jnp.where(p == m1, ii, E), axis=-1, keepdims=True)
    p_excl = jnp.where(ii == i1, _NEG_INF, p)
    m2 = jnp.max(p_excl, axis=-1, keepdims=True)
    i2 = jnp.min(jnp.where(p_excl == m2, ii, E), axis=-1, keepdims=True)

    denom = m1 + m2 + 1e-9
    e0_ref[...] = i1
    e1_ref[...] = i2
    g0_ref[...] = m1 / denom
    g1_ref[...] = m2 / denom

    oh = (ii == i1).astype(jnp.int32) + (ii == i2).astype(jnp.int32)
    h = jnp.sum(oh.reshape(64, T // 64, E), axis=1)
    hist_ref[...] = jnp.concatenate(
        [h, jnp.zeros((64, 16 - E), jnp.int32)], axis=1)


def _gating(x, Wg):
    return pl.pallas_call(
        _gating_body,
        out_shape=[
            jax.ShapeDtypeStruct((T, 1), jnp.int32),
            jax.ShapeDtypeStruct((T, 1), jnp.int32),
            jax.ShapeDtypeStruct((T, 1), jnp.float32),
            jax.ShapeDtypeStruct((T, 1), jnp.float32),
            jax.ShapeDtypeStruct((64, 16), jnp.int32),
        ],
    )(x, Wg)


@functools.lru_cache(maxsize=None)
def _dispatch_kernel(h):
    mesh = plsc.VectorSubcoreMesh(core_axis_name="c", subcore_axis_name="s")
    CH = 32
    base = h * TH

    @functools.partial(
        pl.kernel,
        mesh=mesh,
        out_type=[
            jax.ShapeDtypeStruct((2 * TH // 32, 32), jnp.int32),
            jax.ShapeDtypeStruct((64,), jnp.int32),
            jax.ShapeDtypeStruct((NPADH, D), jnp.float32),
            jax.ShapeDtypeStruct((NPADH, 128), jnp.float32),
        ],
        scratch_types=[
            pltpu.VMEM((128,), jnp.int32),
            pltpu.VMEM((128,), jnp.float32),
            pltpu.VMEM((64, 16), jnp.int32),
            pltpu.VMEM((4, 32), jnp.int32),
            pltpu.VMEM((CH, D), jnp.float32),
            pltpu.VMEM((CH, D), jnp.float32),
            pltpu.VMEM((CH, 128), jnp.float32),
            pltpu.VMEM((CH, 128), jnp.float32),
            pltpu.VMEM((CH, 128), jnp.float32),
            pltpu.VMEM((CH, 128), jnp.float32),
            pltpu.VMEM((64,), jnp.int32),
            pltpu.SemaphoreType.DMA,
            pltpu.SemaphoreType.DMA,
        ],
        compiler_params=pltpu.CompilerParams(needs_layout_passes=False),
    )
    def dispatch(e0_hbm, e1_hbm, g0_hbm, g1_hbm, hist_hbm, x_hbm,
                 pos_hbm, meta_hbm, xs_hbm, gsl_hbm,
                 e_v, g_v, hist_v, pos_v, xA, xB, gA0, gA1, gB0, gB1,
                 meta_v, sem_ld, sem_st):
        wid = lax.axis_index("s") * 2 + lax.axis_index("c")
        tb = base + wid * TPW

        pltpu.sync_copy(e0_hbm.at[pl.ds(tb, TPW)], e_v.at[pl.ds(0, TPW)])
        pltpu.sync_copy(e1_hbm.at[pl.ds(tb, TPW)], e_v.at[pl.ds(TPW, TPW)])
        pltpu.sync_copy(g0_hbm.at[pl.ds(tb, TPW)], g_v.at[pl.ds(0, TPW)])
        pltpu.sync_copy(g1_hbm.at[pl.ds(tb, TPW)], g_v.at[pl.ds(TPW, TPW)])
        pltpu.sync_copy(hist_hbm, hist_v)
        ld = pltpu.async_copy(x_hbm.at[pl.ds(tb, CH)], xA, sem_ld)

        iota16 = lax.iota(jnp.int32, 16)
        tot = jnp.zeros((16,), jnp.int32)
        bef = jnp.zeros((16,), jnp.int32)
        for s in range(32):
            row = hist_v[32 * h + s, :]
            tot = tot + row
            bef = bef + row * (s < wid).astype(jnp.int32)
        padded = jnp.bitwise_and(tot + (MBLK - 1), jnp.int32(-MBLK))
        pbase = plsc.cumsum(padded) - padded
        cur = pbase + bef

        for i in range(2 * TPW // 16):
            v = e_v[pl.ds(i * 16, 16)]
            posv = jnp.zeros((16,), jnp.int32)
            for e in range(E):
                mask = v == e
                mi = mask.astype(jnp.int32)
                rank = plsc.cumsum(mi) - mi
                cur_e = jnp.sum(jnp.where(iota16 == e, cur, 0))
                posv = jnp.where(mask, cur_e + rank, posv)
                cnt = jnp.sum(mi)
                cur = cur + jnp.where(iota16 == e, cnt, 0)
            pos_v[i // 2, pl.ds((i % 2) * 16, 16)] = posv

        pltpu.sync_copy(pos_v.at[pl.ds(0, 2)], pos_hbm.at[pl.ds(wid * 2, 2)])
        pltpu.sync_copy(pos_v.at[pl.ds(2, 2)],
                        pos_hbm.at[pl.ds(64 + wid * 2, 2)])

        @pl.when(wid == 0)
        def _():
            for vb in range(2):
                bidx = iota16 + vb * 16
                acc = jnp.full((16,), -1, jnp.int32)
                for e in range(E):
                    pb_e = jnp.sum(jnp.where(iota16 == e, pbase, 0))
                    acc = acc + (bidx * MBLK >= pb_e).astype(jnp.int32)
                meta_v[pl.ds(vb * 16, 16)] = acc
            meta_v[pl.ds(32, 16)] = jnp.zeros((16,), jnp.int32)
            total = jnp.sum(padded)
            meta_v[pl.ds(48, 16)] = jnp.where(
                iota16 == 0, total // MBLK, 0)
            pltpu.sync_copy(meta_v, meta_hbm)

        def build_gates(gbuf, c, half):
            for i16 in range(CH // 16):
                gvec = g_v[pl.ds(half * TPW + c * CH + i16 * 16, 16)]
                for l in range(16):
                    gbuf[i16 * 16 + l, pl.ds(0, 16)] = jnp.full((16,), gvec[l])

        xbufs = [xA, xB]
        gbufs = [(gA0, gA1), (gB0, gB1)]
        n_chunks = TPW // CH
        build_gates(gA0, 0, 0)
        build_gates(gA1, 0, 1)
        pending = []
        for c in range(n_chunks):
            cur_x = xbufs[c % 2]
            cur_g0, cur_g1 = gbufs[c % 2]
            ld.wait()
            if c + 1 < n_chunks:
                if c >= 1:
                    for _ in range(4):
                        pending.pop(0).wait()
                ld = pltpu.async_copy(
                    x_hbm.at[pl.ds(tb + (c + 1) * CH, CH)], xbufs[(c + 1) % 2],
                    sem_ld)
            i0 = pos_v.at[c]
            i1 = pos_v.at[2 + c]
            pending.append(pltpu.async_copy(cur_x, xs_hbm.at[i0], sem_st))
            pending.append(pltpu.async_copy(cur_x, xs_hbm.at[i1], sem_st))
            pending.append(pltpu.async_copy(cur_g0, gsl_hbm.at[i0], sem_st))
            pending.append(pltpu.async_copy(cur_g1, gsl_hbm.at[i1], sem_st))
            if c + 1 < n_chunks:
                build_gates(gbufs[(c + 1) % 2][0], c + 1, 0)
                build_gates(gbufs[(c + 1) % 2][1], c + 1, 1)
        while pending:
            pending.pop(0).wait()

    return dispatch


def _gemm_body(s_ref, xs_ref, gsl_ref, we_ref, be_ref, o_ref):
    i = pl.program_id(0)
    nblk = s_ref[48]

    @pl.when(i < nblk)
    def _():
        acc = jnp.dot(xs_ref[...].astype(jnp.bfloat16), we_ref[0],
                      preferred_element_type=jnp.float32)
        acc = acc + be_ref[0]
        o_ref[...] = gsl_ref[...][:, 0:1] * acc

    @pl.when(i >= nblk)
    def _():
        o_ref[...] = jnp.zeros_like(o_ref)


def _grouped_gemm(blkmeta, xs, gsl, We16, be3):
    grid_spec = pltpu.PrefetchScalarGridSpec(
        num_scalar_prefetch=1,
        grid=(NBH,),
        in_specs=[
            pl.BlockSpec((MBLK, D), lambda i, s: (i, 0)),
            pl.BlockSpec((MBLK, 128), lambda i, s: (i, 0)),
            pl.BlockSpec((1, D, D), lambda i, s: (s[i], 0, 0)),
            pl.BlockSpec((1, 1, D), lambda i, s: (s[i], 0, 0)),
        ],
        out_specs=pl.BlockSpec((MBLK, D), lambda i, s: (i, 0)),
    )
    return pl.pallas_call(
        _gemm_body,
        grid_spec=grid_spec,
        out_shape=jax.ShapeDtypeStruct((NPADH, D), jnp.float32),
        compiler_params=pltpu.CompilerParams(
            dimension_semantics=("parallel",)),
    )(blkmeta, xs, gsl, We16, be3)


@functools.lru_cache(maxsize=None)
def _combine_kernel(h):
    mesh = plsc.VectorSubcoreMesh(core_axis_name="c", subcore_axis_name="s")

    @functools.partial(
        pl.kernel,
        mesh=mesh,
        out_type=jax.ShapeDtypeStruct((TH, D), jnp.float32),
        scratch_types=[
            pltpu.VMEM((4, 32), jnp.int32),
            pltpu.VMEM((16, D), jnp.float32),
            pltpu.VMEM((16, D), jnp.float32),
            pltpu.VMEM((16, D), jnp.float32),
            pltpu.VMEM((16, D), jnp.float32),
            pltpu.VMEM((16, D), jnp.float32),
            pltpu.VMEM((16, D), jnp.float32),
            pltpu.SemaphoreType.DMA,
            pltpu.SemaphoreType.DMA,
        ],
        compiler_params=pltpu.CompilerParams(needs_layout_passes=False),
    )
    def combine(ys_hbm, pos_hbm, out_hbm,
                p_v, r0A, r1A, r0B, r1B, oA, oB, sem_g, sem_o):
        wid = lax.axis_index("s") * 2 + lax.axis_index("c")
        t0 = wid * TPW
        pltpu.sync_copy(pos_hbm.at[pl.ds(wid * 2, 2)], p_v.at[pl.ds(0, 2)])
        pltpu.sync_copy(pos_hbm.at[pl.ds(64 + wid * 2, 2)],
                        p_v.at[pl.ds(2, 2)])

        rbufs = [(r0A, r1A), (r0B, r1B)]
        obufs = [oA, oB]
        n_chunks = TPW // 16

        def issue_gathers(j, bufs):
            r = j // 2
            off = (j % 2) * 16
            i0 = p_v[r, pl.ds(off, 16)]
            i1 = p_v[2 + r, pl.ds(off, 16)]
            c0 = pltpu.async_copy(ys_hbm.at[i0], bufs[0], sem_g)
            c1 = pltpu.async_copy(ys_hbm.at[i1], bufs[1], sem_g)
            return (c0, c1)

        pend_g = issue_gathers(0, rbufs[0])
        pend_o = [None, None]
        for j in range(n_chunks):
            r0, r1 = rbufs[j % 2]
            ov = obufs[j % 2]
            pend_g[0].wait()
            pend_g[1].wait()
            if j + 1 < n_chunks:
                pend_g = issue_gathers(j + 1, rbufs[(j + 1) % 2])
            if pend_o[j % 2] is not None:
                pend_o[j % 2].wait()

            def _row(r_i, _):
                for dj in range(64):
                    sl = pl.ds(dj * 16, 16)
                    ov[r_i, sl] = r0[r_i, sl] + r1[r_i, sl]
                return 0

            lax.fori_loop(0, 16, _row, 0)
            pend_o[j % 2] = pltpu.async_copy(
                ov, out_hbm.at[pl.ds(t0 + j * 16, 16)], sem_o)
        for po in pend_o:
            if po is not None:
                po.wait()

    return combine


def kernel(x, Wg, We, be):
    e0, e1, g0, g1, hist = _gating(x, Wg)
    e0, e1 = e0.reshape(T), e1.reshape(T)
    g0, g1 = g0.reshape(T), g1.reshape(T)
    We16 = We.astype(jnp.bfloat16)
    be3 = be.reshape(E, 1, D)
    outs = []
    for h in (0, 1):
        pos, blkmeta, xs, gsl = _dispatch_kernel(h)(e0, e1, g0, g1, hist, x)
        ys = _grouped_gemm(blkmeta, xs, gsl, We16, be3)
        outs.append(_combine_kernel(h)(ys, pos))
    return jnp.concatenate(outs, axis=0)


# speedup vs baseline: 1.2291x; 1.2291x over previous
import functools

import jax
import jax.numpy as jnp
from jax import lax
from jax.experimental import pallas as pl
from jax.experimental.pallas import tpu as pltpu
from jax.experimental.pallas import tpu_sc as plsc

T = 4096
D = 1024
E = 8
NSLICE = 32
MBLK = 256
NB = 40
NPAD = NB * MBLK
_NEG_INF = float("-inf")


def _gating_body(x_ref, wg_ref, e_ref, g_ref, hist_ref):
    x = x_ref[...]
    logits = jnp.dot(x, wg_ref[...], preferred_element_type=jnp.float32)
    m = jnp.max(logits, axis=-1, keepdims=True)
    p = jnp.exp(logits - m)
    p = p / jnp.sum(p, axis=-1, keepdims=True)

    ii = lax.broadcasted_iota(jnp.int32, p.shape, 1)
    m1 = jnp.max(p, axis=-1, keepdims=True)
    i1 = jnp.min(jnp.where(p == m1, ii, E), axis=-1, keepdims=True)
    p_excl = jnp.where(ii == i1, _NEG_INF, p)
    m2 = jnp.max(p_excl, axis=-1, keepdims=True)
    i2 = jnp.min(jnp.where(p_excl == m2, ii, E), axis=-1, keepdims=True)

    denom = m1 + m2 + 1e-9
    e_ref[0:T, :] = i1
    e_ref[T:2 * T, :] = i2
    g_ref[0:T, :] = m1 / denom
    g_ref[T:2 * T, :] = m2 / denom

    oh = (ii == i1).astype(jnp.int32) + (ii == i2).astype(jnp.int32)
    h = jnp.sum(oh.reshape(NSLICE, T // NSLICE, E), axis=1)
    hist_ref[...] = jnp.concatenate(
        [h, jnp.zeros((NSLICE, 16 - E), jnp.int32)], axis=1)


def _gating(x, Wg):
    return pl.pallas_call(
        _gating_body,
        out_shape=[
            jax.ShapeDtypeStruct((2 * T, 1), jnp.int32),
            jax.ShapeDtypeStruct((2 * T, 1), jnp.float32),
            jax.ShapeDtypeStruct((NSLICE, 16), jnp.int32),
        ],
    )(x, Wg)


@functools.lru_cache(maxsize=None)
def _dispatch_kernel():
    mesh = plsc.VectorSubcoreMesh(core_axis_name="c", subcore_axis_name="s")
    CH = 32

    @functools.partial(
        pl.kernel,
        mesh=mesh,
        out_type=[
            jax.ShapeDtypeStruct((2 * T // 32, 32), jnp.int32),
            jax.ShapeDtypeStruct((64,), jnp.int32),
            jax.ShapeDtypeStruct((NPAD, D), jnp.float32),
            jax.ShapeDtypeStruct((NPAD, 128), jnp.float32),
        ],
        scratch_types=[
            pltpu.VMEM((256,), jnp.int32),
            pltpu.VMEM((256,), jnp.float32),
            pltpu.VMEM((NSLICE, 16), jnp.int32),
            pltpu.VMEM((8, 32), jnp.int32),
            pltpu.VMEM((CH, D), jnp.float32),
            pltpu.VMEM((CH, D), jnp.float32),
            pltpu.VMEM((CH, 128), jnp.float32),
            pltpu.VMEM((CH, 128), jnp.float32),
            pltpu.VMEM((CH, 128), jnp.float32),
            pltpu.VMEM((CH, 128), jnp.float32),
            pltpu.VMEM((64,), jnp.int32),
            pltpu.SemaphoreType.DMA,
            pltpu.SemaphoreType.DMA,
        ],
        compiler_params=pltpu.CompilerParams(needs_layout_passes=False),
    )
    def dispatch(e_hbm, g_hbm, hist_hbm, x_hbm,
                 pos_hbm, meta_hbm, xs_hbm, gsl_hbm,
                 e_v, g_v, hist_v, pos_v, xA, xB, gA0, gA1, gB0, gB1,
                 meta_v, sem_ld, sem_st):
        wid = lax.axis_index("s") * 2 + lax.axis_index("c")
        tb = wid * 128

        pltpu.sync_copy(e_hbm.at[pl.ds(tb, 128)], e_v.at[pl.ds(0, 128)])
        pltpu.sync_copy(e_hbm.at[pl.ds(T + tb, 128)], e_v.at[pl.ds(128, 128)])
        pltpu.sync_copy(g_hbm.at[pl.ds(tb, 128)], g_v.at[pl.ds(0, 128)])
        pltpu.sync_copy(g_hbm.at[pl.ds(T + tb, 128)], g_v.at[pl.ds(128, 128)])
        pltpu.sync_copy(hist_hbm, hist_v)
        ld = pltpu.async_copy(x_hbm.at[pl.ds(tb, CH)], xA, sem_ld)

        iota16 = lax.iota(jnp.int32, 16)
        tot = jnp.zeros((16,), jnp.int32)
        bef = jnp.zeros((16,), jnp.int32)
        for s in range(NSLICE):
            row = hist_v[s, :]
            tot = tot + row
            bef = bef + row * (s < wid).astype(jnp.int32)
        padded = jnp.bitwise_and(tot + (MBLK - 1), jnp.int32(-MBLK))
        pbase = plsc.cumsum(padded) - padded
        cur = pbase + bef

        for i in range(16):
            v = e_v[pl.ds(i * 16, 16)]
            posv = jnp.zeros((16,), jnp.int32)
            for e in range(E):
                mask = v == e
                mi = mask.astype(jnp.int32)
                rank = plsc.cumsum(mi) - mi
                cur_e = jnp.sum(jnp.where(iota16 == e, cur, 0))
                posv = jnp.where(mask, cur_e + rank, posv)
                cnt = jnp.sum(mi)
                cur = cur + jnp.where(iota16 == e, cnt, 0)
            pos_v[i // 2, pl.ds((i % 2) * 16, 16)] = posv

        pltpu.sync_copy(pos_v, pos_hbm.at[pl.ds(wid * 8, 8)])

        @pl.when(wid == 0)
        def _():
            for vb in range(3):
                bidx = iota16 + vb * 16
                acc = jnp.full((16,), -1, jnp.int32)
                for e in range(E):
                    pb_e = jnp.sum(jnp.where(iota16 == e, pbase, 0))
                    acc = acc + (bidx * MBLK >= pb_e).astype(jnp.int32)
                meta_v[pl.ds(vb * 16, 16)] = acc
            total = jnp.sum(padded)
            meta_v[pl.ds(48, 16)] = jnp.where(
                iota16 == 0, total // MBLK, 0)
            pltpu.sync_copy(meta_v, meta_hbm)

        def build_gates(gbuf, c, half):
            for i16 in range(CH // 16):
                gvec = g_v[pl.ds(half * 128 + c * CH + i16 * 16, 16)]
                for l in range(16):
                    gbuf[i16 * 16 + l, pl.ds(0, 16)] = jnp.full((16,), gvec[l])

        xbufs = [xA, xB]
        gbufs = [(gA0, gA1), (gB0, gB1)]
        n_chunks = 128 // CH
        build_gates(gA0, 0, 0)
        build_gates(gA1, 0, 1)
        pending = []
        for c in range(n_chunks):
            cur_x = xbufs[c % 2]
            cur_g0, cur_g1 = gbufs[c % 2]
            ld.wait()
            if c + 1 < n_chunks:
                if c >= 1:
                    for _ in range(4):
                        pending.pop(0).wait()
                ld = pltpu.async_copy(
                    x_hbm.at[pl.ds(tb + (c + 1) * CH, CH)], xbufs[(c + 1) % 2],
                    sem_ld)
            i0 = pos_v.at[c]
            i1 = pos_v.at[4 + c]
            pending.append(pltpu.async_copy(cur_x, xs_hbm.at[i0], sem_st))
            pending.append(pltpu.async_copy(cur_x, xs_hbm.at[i1], sem_st))
            pending.append(pltpu.async_copy(cur_g0, gsl_hbm.at[i0], sem_st))
            pending.append(pltpu.async_copy(cur_g1, gsl_hbm.at[i1], sem_st))
            if c + 1 < n_chunks:
                build_gates(gbufs[(c + 1) % 2][0], c + 1, 0)
                build_gates(gbufs[(c + 1) % 2][1], c + 1, 1)
        while pending:
            pending.pop(0).wait()

    return dispatch


def _gemm_body(s_ref, xs_ref, gsl_ref, we_ref, be_ref, o_ref):
    i = pl.program_id(0)
    nblk = s_ref[48]

    @pl.when(i < nblk)
    def _():
        acc = jnp.dot(xs_ref[...], we_ref[0],
                      preferred_element_type=jnp.float32)
        acc = acc + be_ref[0]
        o_ref[...] = gsl_ref[...][:, 0:1] * acc

    @pl.when(i >= nblk)
    def _():
        o_ref[...] = jnp.zeros_like(o_ref)


def _grouped_gemm(blkmeta, xs, gsl, We, be):
    grid_spec = pltpu.PrefetchScalarGridSpec(
        num_scalar_prefetch=1,
        grid=(NB,),
        in_specs=[
            pl.BlockSpec((MBLK, D), lambda i, s: (i, 0)),
            pl.BlockSpec((MBLK, 128), lambda i, s: (i, 0)),
            pl.BlockSpec((1, D, D), lambda i, s: (s[i], 0, 0)),
            pl.BlockSpec((1, 1, D), lambda i, s: (s[i], 0, 0)),
        ],
        out_specs=pl.BlockSpec((MBLK, D), lambda i, s: (i, 0)),
    )
    return pl.pallas_call(
        _gemm_body,
        grid_spec=grid_spec,
        out_shape=jax.ShapeDtypeStruct((NPAD, D), jnp.float32),
    )(blkmeta, xs, gsl, We, be.reshape(E, 1, D))


@functools.lru_cache(maxsize=None)
def _combine_kernel():
    mesh = plsc.VectorSubcoreMesh(core_axis_name="c", subcore_axis_name="s")

    @functools.partial(
        pl.kernel,
        mesh=mesh,
        out_type=jax.ShapeDtypeStruct((T, D), jnp.float32),
        scratch_types=[
            pltpu.VMEM((8, 32), jnp.int32),
            pltpu.VMEM((16, D), jnp.float32),
            pltpu.VMEM((16, D), jnp.float32),
            pltpu.VMEM((16, D), jnp.float32),
            pltpu.VMEM((16, D), jnp.float32),
            pltpu.VMEM((16, D), jnp.float32),
            pltpu.VMEM((16, D), jnp.float32),
            pltpu.SemaphoreType.DMA,
            pltpu.SemaphoreType.DMA,
        ],
        compiler_params=pltpu.CompilerParams(needs_layout_passes=False),
    )
    def combine(ys_hbm, pos_hbm, out_hbm,
                p_v, r0A, r1A, r0B, r1B, oA, oB, sem_g, sem_o):
        wid = lax.axis_index("s") * 2 + lax.axis_index("c")
        t0 = wid * 128
        pltpu.sync_copy(pos_hbm.at[pl.ds(wid * 8, 8)], p_v)

        rbufs = [(r0A, r1A), (r0B, r1B)]
        obufs = [oA, oB]

        def issue_gathers(j, bufs):
            r = j // 2
            off = (j % 2) * 16
            i0 = p_v[r, pl.ds(off, 16)]
            i1 = p_v[4 + r, pl.ds(off, 16)]
            c0 = pltpu.async_copy(ys_hbm.at[i0], bufs[0], sem_g)
            c1 = pltpu.async_copy(ys_hbm.at[i1], bufs[1], sem_g)
            return (c0, c1)

        pend_g = issue_gathers(0, rbufs[0])
        pend_o = [None, None]
        for j in range(8):
            r0, r1 = rbufs[j % 2]
            ov = obufs[j % 2]
            pend_g[0].wait()
            pend_g[1].wait()
            if j + 1 < 8:
                pend_g = issue_gathers(j + 1, rbufs[(j + 1) % 2])
            if pend_o[j % 2] is not None:
                pend_o[j % 2].wait()

            def _row(r_i, _):
                for dj in range(64):
                    sl = pl.ds(dj * 16, 16)
                    ov[r_i, sl] = r0[r_i, sl] + r1[r_i, sl]
                return 0

            lax.fori_loop(0, 16, _row, 0)
            pend_o[j % 2] = pltpu.async_copy(
                ov, out_hbm.at[pl.ds(t0 + j * 16, 16)], sem_o)
        for po in pend_o:
            if po is not None:
                po.wait()

    return combine


def kernel(x, Wg, We, be):
    e_pair, g_pair, hist = _gating(x, Wg)
    pos, blkmeta, xs, gsl = _dispatch_kernel()(
        e_pair.reshape(2 * T), g_pair.reshape(2 * T), hist, x)
    ys = _grouped_gemm(blkmeta, xs, gsl, We, be)
    return _combine_kernel()(ys, pos)
